# baseline (speedup 1.0000x reference)
import jax
import jax.numpy as jnp
from jax.experimental import pallas as pl
from jax.experimental.pallas import tpu as pltpu
from jax.experimental.pallas import tpu_sc as plsc

_LANES = 16
_BR = 8
_BC = 256


def kernel(x, table):
    B, L, D = x.shape
    xf = x.reshape(B * L, D)
    mesh = plsc.VectorSubcoreMesh(core_axis_name="core",
                                  subcore_axis_name="subcore")

    @pl.kernel(out_type=jax.ShapeDtypeStruct((B * L, D), x.dtype), mesh=mesh)
    def sc_add(x_hbm, t_hbm, o_hbm):
        def body(x_vmem, t_vmem, o_vmem):
            @pl.loop(0, _BR)
            def _(r):
                @pl.loop(0, _BC, step=4 * _LANES)
                def _(c):
                    for u in range(4):
                        s = (r, pl.ds(c + u * _LANES, _LANES))
                        o_vmem.at[*s][...] = (
                            x_vmem.at[*s][...] + t_vmem.at[*s][...]
                        )

        n_tab_blocks = L // _BR
        pltpu.emit_pipeline(
            body,
            grid=(B * L // _BR, D // _BC),
            in_specs=[
                pl.BlockSpec((_BR, _BC), index_map=lambda i, j: (i, j)),
                pl.BlockSpec((_BR, _BC),
                             index_map=lambda i, j: (i % n_tab_blocks, j)),
            ],
            out_specs=[pl.BlockSpec((_BR, _BC), index_map=lambda i, j: (i, j))],
            core_axis_name=("core", "subcore"),
            dimension_semantics=(pltpu.PARALLEL, pltpu.PARALLEL),
        )(x_hbm, t_hbm, o_hbm)

    return sc_add(xf, table).reshape(B, L, D)


# speedup vs baseline: 4.8851x; 4.8851x over previous
import jax
import jax.numpy as jnp
from jax.experimental import pallas as pl
from jax.experimental.pallas import tpu as pltpu


def _add_kernel(x_ref, t_ref, o_ref):
    o_ref[...] = x_ref[...] + t_ref[...]


def kernel(x, table):
    B, L, D = x.shape
    BL = 512
    return pl.pallas_call(
        _add_kernel,
        grid=(L // BL,),
        in_specs=[
            pl.BlockSpec((B, BL, D), lambda i: (0, i, 0)),
            pl.BlockSpec((BL, D), lambda i: (i, 0)),
        ],
        out_specs=pl.BlockSpec((B, BL, D), lambda i: (0, i, 0)),
        out_shape=jax.ShapeDtypeStruct(x.shape, x.dtype),
        compiler_params=pltpu.CompilerParams(
            dimension_semantics=("parallel",),
        ),
    )(x, table)
